# baseline (speedup 1.0000x reference)
import jax
import jax.numpy as jnp
from jax.experimental import pallas as pl
from jax.experimental.pallas import tpu as pltpu

B_ = 1024
S_ = 50
NUM_CLASSES_ = 1000
CBLK_ = 200


def _onehot_block(x_ref, o_ref):
    j = pl.program_id(1)
    ids = jax.lax.broadcasted_iota(jnp.int32, o_ref.shape, 1) + j * CBLK_
    o_ref[...] = (ids == x_ref[...]).astype(o_ref.dtype)


def kernel(x):
    out_dtype = jnp.zeros((), jnp.int64).dtype
    xt = jnp.transpose(x).astype(jnp.int32).reshape(S_, 1, B_)
    out_t = pl.pallas_call(
        _onehot_block,
        grid=(S_, NUM_CLASSES_ // CBLK_),
        in_specs=[pl.BlockSpec((1, 1, B_), lambda i, j: (i, 0, 0))],
        out_specs=pl.BlockSpec((1, CBLK_, B_), lambda i, j: (i, j, 0)),
        out_shape=jax.ShapeDtypeStruct((S_, NUM_CLASSES_, B_), out_dtype),
    )(xt)
    return jnp.transpose(out_t, (2, 0, 1))


# speedup vs baseline: 2.1898x; 2.1898x over previous
import jax
import jax.numpy as jnp
from jax.experimental import pallas as pl
from jax.experimental.pallas import tpu as pltpu

B_ = 1024
S_ = 50
NUM_CLASSES_ = 1000


def _onehot_block(x_ref, o_ref):
    i = pl.program_id(0)
    ids = jax.lax.broadcasted_iota(jnp.int32, (NUM_CLASSES_, B_), 0)
    xv = x_ref[pl.ds(i, 1), :]
    o_ref[...] = (ids == xv).astype(o_ref.dtype)[None]


def kernel(x):
    out_dtype = jnp.zeros((), jnp.int64).dtype
    xt = jnp.transpose(x).astype(jnp.int32)
    out_t = pl.pallas_call(
        _onehot_block,
        grid=(S_,),
        in_specs=[pl.BlockSpec(memory_space=pltpu.MemorySpace.VMEM)],
        out_specs=pl.BlockSpec((1, NUM_CLASSES_, B_), lambda i: (i, 0, 0)),
        out_shape=jax.ShapeDtypeStruct((S_, NUM_CLASSES_, B_), out_dtype),
    )(xt)
    return jnp.transpose(out_t, (2, 0, 1))
